# baseline (speedup 1.0000x reference)
import jax
import jax.numpy as jnp
from jax.experimental import pallas as pl
from jax.experimental.pallas import tpu as pltpu

N, C, H, W = 16, 512, 32, 32
CH = 16
NCH = (N * H) // CH
K = 8
D = 6


def _body(mask_ref, w1_ref, b1_ref, w2_ref, b2_ref, wfc_ref, bfc_ref,
          lst_hbm, gui_hbm,
          m_ref, out_l_hbm, out_g_hbm,
          lbuf, gbuf, sel_ref, lin, gin, lout, gout):
    def in_copy(i, slot):
        return (
            pltpu.make_async_copy(lst_hbm.at[pl.ds(i * CH, CH)],
                                  lbuf.at[slot], lin.at[slot]),
            pltpu.make_async_copy(gui_hbm.at[pl.ds(i * CH, CH)],
                                  gbuf.at[slot], gin.at[slot]),
        )

    def out_copy(i, slot):
        return (
            pltpu.make_async_copy(lbuf.at[slot],
                                  out_l_hbm.at[pl.ds(i * CH, CH)],
                                  lout.at[slot]),
            pltpu.make_async_copy(gbuf.at[slot],
                                  out_g_hbm.at[pl.ds(i * CH, CH)],
                                  gout.at[slot]),
        )

    for k in range(D):
        a, b = in_copy(k, k)
        a.start()
        b.start()

    h = jax.nn.relu(
        jnp.dot(mask_ref[...], w1_ref[...],
                preferred_element_type=jnp.float32) + b1_ref[...])
    m = jax.nn.sigmoid(
        jnp.dot(h, w2_ref[...],
                preferred_element_type=jnp.float32) + b2_ref[...])
    s = jax.nn.sigmoid(
        jnp.dot(m, wfc_ref[...],
                preferred_element_type=jnp.float32) + bfc_ref[...])
    m_ref[...] = m
    sel_ref[...] = s

    for i in range(NCH):
        slot = i % K
        a, b = in_copy(i, slot)
        a.wait()
        b.wait()
        n = (i * CH) // H
        cond = (sel_ref[n, :] > 0.5)[None, None, :]
        l = lbuf[slot]
        g = gbuf[slot]
        lbuf[slot] = jnp.where(cond, g, l)
        gbuf[slot] = jnp.where(cond, l, g)
        a, b = out_copy(i, slot)
        a.start()
        b.start()
        pf = i + D
        if pf < NCH:
            s2 = pf % K
            if pf >= K:
                a, b = out_copy(pf - K, s2)
                a.wait()
                b.wait()
            a, b = in_copy(pf, s2)
            a.start()
            b.start()

    for s2 in range(K):
        a, b = out_copy(NCH - K + s2, (NCH - K + s2) % K)
        a.wait()
        b.wait()


def kernel(lst, gui, mask, W1, b1, W2, b2, Wfc, bfc):
    lst3 = lst.transpose(0, 2, 3, 1).reshape(N * H, W, C)
    gui3 = gui.transpose(0, 2, 3, 1).reshape(N * H, W, C)

    m, out_l3, out_g3 = pl.pallas_call(
        _body,
        in_specs=[
            pl.BlockSpec(memory_space=pltpu.VMEM),
            pl.BlockSpec(memory_space=pltpu.VMEM),
            pl.BlockSpec(memory_space=pltpu.VMEM),
            pl.BlockSpec(memory_space=pltpu.VMEM),
            pl.BlockSpec(memory_space=pltpu.VMEM),
            pl.BlockSpec(memory_space=pltpu.VMEM),
            pl.BlockSpec(memory_space=pltpu.VMEM),
            pl.BlockSpec(memory_space=pl.ANY),
            pl.BlockSpec(memory_space=pl.ANY),
        ],
        out_specs=[
            pl.BlockSpec(memory_space=pltpu.VMEM),
            pl.BlockSpec(memory_space=pl.ANY),
            pl.BlockSpec(memory_space=pl.ANY),
        ],
        out_shape=(
            jax.ShapeDtypeStruct((N, C), jnp.float32),
            jax.ShapeDtypeStruct((N * H, W, C), jnp.float32),
            jax.ShapeDtypeStruct((N * H, W, C), jnp.float32),
        ),
        scratch_shapes=[
            pltpu.VMEM((K, CH, W, C), jnp.float32),
            pltpu.VMEM((K, CH, W, C), jnp.float32),
            pltpu.VMEM((N, C), jnp.float32),
            pltpu.SemaphoreType.DMA((K,)),
            pltpu.SemaphoreType.DMA((K,)),
            pltpu.SemaphoreType.DMA((K,)),
            pltpu.SemaphoreType.DMA((K,)),
        ],
        compiler_params=pltpu.CompilerParams(vmem_limit_bytes=50 * 1024 * 1024),
    )(mask, W1, b1.reshape(1, C), W2, b2.reshape(1, C),
      Wfc, bfc.reshape(1, C), lst3, gui3)

    out_lst = out_l3.reshape(N, H, W, C).transpose(0, 3, 1, 2)
    out_gui = out_g3.reshape(N, H, W, C).transpose(0, 3, 1, 2)
    return (out_lst, out_gui, m)


# speedup vs baseline: 1.0150x; 1.0150x over previous
import jax
import jax.numpy as jnp
from jax.experimental import pallas as pl
from jax.experimental.pallas import tpu as pltpu

N, C, H, W = 16, 512, 32, 32
CH = 32
NCH = (N * H) // CH
K = 6
D = 4


def _body(mask_ref, w1_ref, b1_ref, w2_ref, b2_ref, wfc_ref, bfc_ref,
          lst_hbm, gui_hbm,
          m_ref, out_l_hbm, out_g_hbm,
          lbuf, gbuf, sel_ref, lin, gin, lout, gout):
    def in_copy(i, slot):
        return (
            pltpu.make_async_copy(lst_hbm.at[pl.ds(i * CH, CH)],
                                  lbuf.at[slot], lin.at[slot]),
            pltpu.make_async_copy(gui_hbm.at[pl.ds(i * CH, CH)],
                                  gbuf.at[slot], gin.at[slot]),
        )

    def out_copy(i, slot):
        return (
            pltpu.make_async_copy(lbuf.at[slot],
                                  out_l_hbm.at[pl.ds(i * CH, CH)],
                                  lout.at[slot]),
            pltpu.make_async_copy(gbuf.at[slot],
                                  out_g_hbm.at[pl.ds(i * CH, CH)],
                                  gout.at[slot]),
        )

    for k in range(D):
        a, b = in_copy(k, k)
        a.start()
        b.start()

    h = jax.nn.relu(
        jnp.dot(mask_ref[...], w1_ref[...],
                preferred_element_type=jnp.float32) + b1_ref[...])
    m = jax.nn.sigmoid(
        jnp.dot(h, w2_ref[...],
                preferred_element_type=jnp.float32) + b2_ref[...])
    s = jax.nn.sigmoid(
        jnp.dot(m, wfc_ref[...],
                preferred_element_type=jnp.float32) + bfc_ref[...])
    m_ref[...] = m
    sel_ref[...] = s

    for i in range(NCH):
        slot = i % K
        a, b = in_copy(i, slot)
        a.wait()
        b.wait()
        n = (i * CH) // H
        cond = (sel_ref[n, :] > 0.5)[None, None, :]
        l = lbuf[slot]
        g = gbuf[slot]
        lbuf[slot] = jnp.where(cond, g, l)
        gbuf[slot] = jnp.where(cond, l, g)
        a, b = out_copy(i, slot)
        a.start()
        b.start()
        pf = i + D
        if pf < NCH:
            s2 = pf % K
            if pf >= K:
                a, b = out_copy(pf - K, s2)
                a.wait()
                b.wait()
            a, b = in_copy(pf, s2)
            a.start()
            b.start()

    for s2 in range(K):
        a, b = out_copy(NCH - K + s2, (NCH - K + s2) % K)
        a.wait()
        b.wait()


def kernel(lst, gui, mask, W1, b1, W2, b2, Wfc, bfc):
    lst3 = lst.transpose(0, 2, 3, 1).reshape(N * H, W, C)
    gui3 = gui.transpose(0, 2, 3, 1).reshape(N * H, W, C)

    m, out_l3, out_g3 = pl.pallas_call(
        _body,
        in_specs=[
            pl.BlockSpec(memory_space=pltpu.VMEM),
            pl.BlockSpec(memory_space=pltpu.VMEM),
            pl.BlockSpec(memory_space=pltpu.VMEM),
            pl.BlockSpec(memory_space=pltpu.VMEM),
            pl.BlockSpec(memory_space=pltpu.VMEM),
            pl.BlockSpec(memory_space=pltpu.VMEM),
            pl.BlockSpec(memory_space=pltpu.VMEM),
            pl.BlockSpec(memory_space=pl.ANY),
            pl.BlockSpec(memory_space=pl.ANY),
        ],
        out_specs=[
            pl.BlockSpec(memory_space=pltpu.VMEM),
            pl.BlockSpec(memory_space=pl.ANY),
            pl.BlockSpec(memory_space=pl.ANY),
        ],
        out_shape=(
            jax.ShapeDtypeStruct((N, C), jnp.float32),
            jax.ShapeDtypeStruct((N * H, W, C), jnp.float32),
            jax.ShapeDtypeStruct((N * H, W, C), jnp.float32),
        ),
        scratch_shapes=[
            pltpu.VMEM((K, CH, W, C), jnp.float32),
            pltpu.VMEM((K, CH, W, C), jnp.float32),
            pltpu.VMEM((N, C), jnp.float32),
            pltpu.SemaphoreType.DMA((K,)),
            pltpu.SemaphoreType.DMA((K,)),
            pltpu.SemaphoreType.DMA((K,)),
            pltpu.SemaphoreType.DMA((K,)),
        ],
        compiler_params=pltpu.CompilerParams(vmem_limit_bytes=50 * 1024 * 1024),
    )(mask, W1, b1.reshape(1, C), W2, b2.reshape(1, C),
      Wfc, bfc.reshape(1, C), lst3, gui3)

    out_lst = out_l3.reshape(N, H, W, C).transpose(0, 3, 1, 2)
    out_gui = out_g3.reshape(N, H, W, C).transpose(0, 3, 1, 2)
    return (out_lst, out_gui, m)
